# baseline (speedup 1.0000x reference)
import jax
import jax.numpy as jnp
from jax import lax
from jax.experimental import pallas as pl
from jax.experimental.pallas import tpu as pltpu
from jax.experimental.pallas import tpu_sc as plsc

N = 10000
E = 320000
D = 128
NC = 2
NS = 16
NW = NC * NS
CHUNK = 128
EP = 2560 * CHUNK
CW = (EP // CHUNK) // NW
NPAD = N + 16
RPS = 624
RPS_LAST = N - 15 * RPS
NB1 = 2
NB2 = 4

_mesh = plsc.VectorSubcoreMesh(
    core_axis_name="c", subcore_axis_name="s", num_cores=NC, num_subcores=NS
)


def _sc_segsum_wide(x_hbm, src_hbm, dst_hbm, out_hbm, acc, sidx,
                    didx0, didx1, didx2, didx3,
                    rows0, rows1, sem0, sem1, dsem0, dsem1, dsem2, dsem3):
    c = lax.axis_index("c")
    s = lax.axis_index("s")
    wid = s * NC + c
    c0 = wid * CW

    zero16 = jnp.zeros((16,), jnp.float32)

    def _zrow(i, carry):
        for k in range(D // 16):
            rows0[i, pl.ds(k * 16, 16)] = zero16
        return carry

    lax.fori_loop(0, CHUNK, _zrow, 0)
    r0 = s * RPS

    @pl.when(s < NS - 1)
    def _():
        off = 0
        for m in (128, 128, 128, 128, RPS - 4 * 128):
            pltpu.sync_copy(rows0.at[pl.ds(0, m)],
                            acc.at[pl.ds(r0 + off, m)])
            off += m

    @pl.when(s == NS - 1)
    def _():
        for k in range(5):
            pltpu.sync_copy(rows0, acc.at[pl.ds(15 * RPS + k * CHUNK, CHUNK)])
        pltpu.sync_copy(rows0.at[pl.ds(0, NPAD - N)],
                        acc.at[pl.ds(N, NPAD - N)])

    pltpu.sync_copy(src_hbm.at[pl.ds(c0, CW)], sidx)
    plsc.subcore_barrier()

    e0 = c0 * CHUNK
    rows = (rows0, rows1)
    sems = (sem0, sem1)
    didxs = (didx0, didx1, didx2, didx3)
    dsems = (dsem0, dsem1, dsem2, dsem3)
    NBD = 4
    gpend = [None] * NB1
    dpend = [None] * NBD
    for g in range(NB1):
        gpend[g] = pltpu.async_copy(x_hbm.at[sidx.at[g]], rows[g], sems[g])
    for g in range(NBD):
        dpend[g] = pltpu.async_copy(
            dst_hbm.at[pl.ds(e0 + g * CHUNK, CHUNK)], didxs[g], dsems[g])
    for g in range(CW):
        b = g % NB1
        d = g % NBD
        gpend[b].wait()
        dpend[d].wait()
        pltpu.sync_copy(rows[b], acc.at[didxs[d]], add=True)
        nxt = g + NB1
        if nxt < CW:
            gpend[b] = pltpu.async_copy(x_hbm.at[sidx.at[nxt]], rows[b],
                                        sems[b])
        nxtd = g + NBD
        if nxtd < CW:
            dpend[d] = pltpu.async_copy(
                dst_hbm.at[pl.ds(e0 + nxtd * CHUNK, CHUNK)], didxs[d],
                dsems[d])

    plsc.subcore_barrier()

    @pl.when(s < NS - 1)
    def _():
        pltpu.sync_copy(acc.at[pl.ds(r0, RPS)], out_hbm.at[c, pl.ds(r0, RPS)])

    @pl.when(s == NS - 1)
    def _():
        pltpu.sync_copy(acc.at[pl.ds(15 * RPS, RPS_LAST)],
                        out_hbm.at[c, pl.ds(15 * RPS, RPS_LAST)])


_sc1 = pl.kernel(
    _sc_segsum_wide,
    out_type=jax.ShapeDtypeStruct((NC, N, D), jnp.float32),
    mesh=_mesh,
    scratch_types=[
        pltpu.VMEM_SHARED((NPAD, D), jnp.float32),
        pltpu.VMEM((CW, CHUNK), jnp.int32),
        pltpu.VMEM((CHUNK,), jnp.int32),
        pltpu.VMEM((CHUNK,), jnp.int32),
        pltpu.VMEM((CHUNK,), jnp.int32),
        pltpu.VMEM((CHUNK,), jnp.int32),
        pltpu.VMEM((CHUNK, D), jnp.float32),
        pltpu.VMEM((CHUNK, D), jnp.float32),
        pltpu.SemaphoreType.DMA,
        pltpu.SemaphoreType.DMA,
        pltpu.SemaphoreType.DMA,
        pltpu.SemaphoreType.DMA,
        pltpu.SemaphoreType.DMA,
        pltpu.SemaphoreType.DMA,
    ],
)


def _sc_segsum_scalar(y_hbm, src_hbm, dst_hbm, out_hbm, acc, sidx, didx,
                      yv0, yv1, yv2, yv3, zbuf, sem0, sem1, sem2, sem3):
    c = lax.axis_index("c")
    s = lax.axis_index("s")
    wid = s * NC + c
    c0 = wid * CW

    zero16 = jnp.zeros((16,), jnp.float32)

    def _z(i, carry):
        zbuf[pl.ds(i * 16, 16)] = zero16
        return carry

    lax.fori_loop(0, RPS_LAST // 16, _z, 0)

    @pl.when(s < NS - 1)
    def _():
        pltpu.sync_copy(zbuf.at[pl.ds(0, RPS)], acc.at[pl.ds(s * RPS, RPS)])

    @pl.when(s == NS - 1)
    def _():
        pltpu.sync_copy(zbuf, acc.at[pl.ds(15 * RPS, RPS_LAST)])
        pltpu.sync_copy(zbuf.at[pl.ds(0, NPAD - N)], acc.at[pl.ds(N, NPAD - N)])

    pltpu.sync_copy(src_hbm.at[pl.ds(c0, CW)], sidx)
    pltpu.sync_copy(dst_hbm.at[pl.ds(c0, CW)], didx)
    plsc.subcore_barrier()

    yvs = (yv0, yv1, yv2, yv3)
    sems = (sem0, sem1, sem2, sem3)
    pend = [None] * NB2
    for g in range(NB2):
        pend[g] = pltpu.async_copy(y_hbm.at[sidx.at[g]], yvs[g], sems[g])
    for g in range(CW):
        b = g % NB2
        pend[b].wait()
        pltpu.sync_copy(yvs[b], acc.at[didx.at[g]], add=True)
        nxt = g + NB2
        if nxt < CW:
            pend[b] = pltpu.async_copy(y_hbm.at[sidx.at[nxt]], yvs[b],
                                       sems[b])

    plsc.subcore_barrier()

    @pl.when(s < NS - 1)
    def _():
        pltpu.sync_copy(acc.at[pl.ds(s * RPS, RPS)], zbuf.at[pl.ds(0, RPS)])
        pltpu.sync_copy(zbuf.at[pl.ds(0, RPS)],
                        out_hbm.at[pl.ds(c * N + s * RPS, RPS)])

    @pl.when(s == NS - 1)
    def _():
        pltpu.sync_copy(acc.at[pl.ds(15 * RPS, RPS_LAST)], zbuf)
        pltpu.sync_copy(zbuf,
                        out_hbm.at[pl.ds(c * N + 15 * RPS, RPS_LAST)])


_sc2 = pl.kernel(
    _sc_segsum_scalar,
    out_type=jax.ShapeDtypeStruct((NC * N,), jnp.float32),
    mesh=_mesh,
    scratch_types=[
        pltpu.VMEM_SHARED((NPAD,), jnp.float32),
        pltpu.VMEM((CW, CHUNK), jnp.int32),
        pltpu.VMEM((CW, CHUNK), jnp.int32),
        pltpu.VMEM((CHUNK,), jnp.float32),
        pltpu.VMEM((CHUNK,), jnp.float32),
        pltpu.VMEM((CHUNK,), jnp.float32),
        pltpu.VMEM((CHUNK,), jnp.float32),
        pltpu.VMEM((RPS_LAST,), jnp.float32),
        pltpu.SemaphoreType.DMA,
        pltpu.SemaphoreType.DMA,
        pltpu.SemaphoreType.DMA,
        pltpu.SemaphoreType.DMA,
    ],
)

_BM = 1000


def _tc_dense_body(p0, p1, x, w1rel, w1root, b1, w2rel_t, w2root_t,
                   y_out, r2_out):
    agg = p0[...] + p1[...]
    h = jnp.dot(agg, w1rel[...], preferred_element_type=jnp.float32)
    h = h + jnp.dot(x[...], w1root[...], preferred_element_type=jnp.float32)
    h = jnp.maximum(h + b1[...], 0.0)
    y_out[...] = jnp.sum(h * w2rel_t[...], axis=1, keepdims=True)
    r2_out[...] = jnp.sum(h * w2root_t[...], axis=1, keepdims=True)


def _tc_out_body(s0, s1, r2, b2, o):
    o[...] = jax.nn.sigmoid(s0[...] + s1[...] + r2[...] + b2[...])


def kernel(x, edge_index, W1_rel, W1_root, b1, W2_rel, W2_root, b2):
    pad = EP - E
    src = jnp.concatenate([edge_index[0], jnp.zeros((pad,), jnp.int32)])
    dst = jnp.concatenate([edge_index[1], jnp.full((pad,), N, jnp.int32)])
    src2d = src.reshape(EP // CHUNK, CHUNK)
    dst2d = dst.reshape(EP // CHUNK, CHUNK)

    parts = _sc1(x, src2d, dst)

    full = pl.BlockSpec((D, D), lambda i: (0, 0))
    row1 = pl.BlockSpec((1, D), lambda i: (0, 0))
    blk = pl.BlockSpec((_BM, D), lambda i: (i, 0))
    col = pl.BlockSpec((_BM, 1), lambda i: (i, 0))
    y, r2 = pl.pallas_call(
        _tc_dense_body,
        grid=(N // _BM,),
        in_specs=[blk, blk, blk, full, full, row1, row1, row1],
        out_specs=[col, col],
        out_shape=[
            jax.ShapeDtypeStruct((N, 1), jnp.float32),
            jax.ShapeDtypeStruct((N, 1), jnp.float32),
        ],
    )(parts[0], parts[1], x, W1_rel, W1_root, b1.reshape(1, D),
      W2_rel.reshape(1, D), W2_root.reshape(1, D))

    sparts = _sc2(y.reshape(N), src2d, dst2d)

    one = pl.BlockSpec((1, 1), lambda i: (0, 0))
    out = pl.pallas_call(
        _tc_out_body,
        grid=(N // _BM,),
        in_specs=[col, col, col, one],
        out_specs=col,
        out_shape=jax.ShapeDtypeStruct((N, 1), jnp.float32),
    )(sparts[:N].reshape(N, 1), sparts[N:].reshape(N, 1), r2,
      b2.reshape(1, 1))
    return out
